# baseline (speedup 1.0000x reference)
import functools

import jax
import jax.numpy as jnp
from jax.experimental import pallas as pl
from jax.experimental.pallas import tpu as pltpu


def _kern(scr1_ref, scr2_ref, fmap_ref, out_ref, cnt_ref, bval_ref,
          bfeat_ref, *, nhb, hb_sz, wf):
    hb = pl.program_id(1)
    hb_n = hb_sz
    ws = 4 * wf
    a = scr1_ref[0]
    b = scr2_ref[0]
    i_q = a.shape[0]
    t = a + b
    tsh = jnp.concatenate([t[:, :, 1:], t[:, :, :1]], axis=2)
    r = t + tsh
    lane = jax.lax.broadcasted_iota(jnp.int32, (i_q, hb_n, ws), 2)
    valid = (lane % 4) == 1
    mask_full = jnp.where(jnp.logical_and(r > 2.0, valid), 1.0, 0.0)

    k_io = jax.lax.broadcasted_iota(jnp.int32, (ws, wf), 0)
    j_io = jax.lax.broadcasted_iota(jnp.int32, (ws, wf), 1)
    sel = (k_io == 4 * j_io + 1).astype(jnp.float32)
    maskd = jax.lax.dot_general(
        mask_full.reshape(i_q * hb_n, ws), sel,
        (((1,), (0,)), ((), ())), precision=jax.lax.Precision.DEFAULT,
        preferred_element_type=jnp.float32).reshape(i_q, hb_n * wf)

    fm = fmap_ref[0]
    partial = jax.lax.dot_general(
        maskd, fm, (((1,), (1,)), ((), ())),
        precision=jax.lax.Precision.DEFAULT,
        preferred_element_type=jnp.float32)
    cnt_blk = jnp.sum(maskd, axis=1, keepdims=True)

    vm = jnp.where(valid, r, -1.0)
    bmax = jnp.max(vm, axis=(1, 2))[:, None]
    h_io = jax.lax.broadcasted_iota(jnp.int32, (i_q, hb_n, ws), 1)
    gidx = ((hb * hb_n + h_io) * wf + jax.lax.shift_right_logical(lane - 1, 2))
    cand = jnp.where(jnp.logical_and(valid, r == bmax[:, :, None]), gidx,
                     jnp.int32(2147483647))
    fidx = jnp.min(cand, axis=(1, 2))[:, None]
    sidx = (jax.lax.broadcasted_iota(jnp.int32, (i_q, hb_n * wf), 1)
            + hb * (hb_n * wf))
    onehot = (sidx == fidx).astype(jnp.float32)
    bfeat_blk = jax.lax.dot_general(onehot, fm, (((1,), (1,)), ((), ())),
                                    precision=jax.lax.Precision.DEFAULT,
                                    preferred_element_type=jnp.float32)

    @pl.when(hb == 0)
    def _init():
        out_ref[0] = partial
        cnt_ref[...] = cnt_blk
        bval_ref[...] = bmax
        bfeat_ref[...] = bfeat_blk

    @pl.when(hb != 0)
    def _acc():
        out_ref[0] += partial
        cnt_ref[...] += cnt_blk
        upd = bmax > bval_ref[...]
        bval_ref[...] = jnp.where(upd, bmax, bval_ref[...])
        bfeat_ref[...] = jnp.where(upd, bfeat_blk, bfeat_ref[...])

    @pl.when(hb == nhb - 1)
    def _fin():
        cnt = cnt_ref[...]
        out_ref[0] = jnp.where(cnt > 0.0, out_ref[0] / cnt, bfeat_ref[...])


def kernel(features, scribbles):
    fmap = features[-1]
    b, c, hf, wf = fmap.shape
    i_q = scribbles.shape[1]
    assert scribbles.shape[2] == 4 * hf and scribbles.shape[3] == 4 * wf
    scr_v = scribbles.reshape(b, i_q, hf, 4 * 4 * wf)
    fmap_v = fmap.reshape(b, c, hf * wf)

    hb_sz = 16
    nhb = hf // hb_sz
    out = pl.pallas_call(
        functools.partial(_kern, nhb=nhb, hb_sz=hb_sz, wf=wf),
        grid=(b, nhb),
        in_specs=[
            pl.BlockSpec((1, i_q, hb_sz, 4 * wf),
                         lambda bb, hh: (bb, 0, hh, 1)),
            pl.BlockSpec((1, i_q, hb_sz, 4 * wf),
                         lambda bb, hh: (bb, 0, hh, 2)),
            pl.BlockSpec((1, c, hb_sz * wf),
                         lambda bb, hh: (bb, 0, hh)),
        ],
        out_specs=pl.BlockSpec((1, i_q, c), lambda bb, hh: (bb, 0, 0)),
        out_shape=jax.ShapeDtypeStruct((b, i_q, c), jnp.float32),
        scratch_shapes=[
            pltpu.VMEM((i_q, 1), jnp.float32),
            pltpu.VMEM((i_q, 1), jnp.float32),
            pltpu.VMEM((i_q, c), jnp.float32),
        ],
    )(scr_v, scr_v, fmap_v)
    return out


# speedup vs baseline: 1.4572x; 1.4572x over previous
import functools

import jax
import jax.numpy as jnp
from jax.experimental import pallas as pl
from jax.experimental.pallas import tpu as pltpu


def _kern(scr_ref, fmap_ref, out_ref, cnt_ref, bval_ref, bfeat_ref, *,
          nhb, hb_sz, wf):
    hb = pl.program_id(1)
    hb_n = hb_sz
    ws = 4 * wf
    scr = scr_ref[0]
    i_q = scr.shape[0]
    scr4 = scr.reshape(i_q, hb_n, 4, ws)
    t = scr4[:, :, 1, :] + scr4[:, :, 2, :]
    tsh = jnp.concatenate([t[:, :, 1:], t[:, :, :1]], axis=2)
    r = t + tsh
    lane = jax.lax.broadcasted_iota(jnp.int32, (i_q, hb_n, ws), 2)
    valid = (lane % 4) == 1
    mask_full = jnp.where(jnp.logical_and(r > 2.0, valid), 1.0, 0.0)

    k_io = jax.lax.broadcasted_iota(jnp.int32, (ws, wf), 0)
    j_io = jax.lax.broadcasted_iota(jnp.int32, (ws, wf), 1)
    sel = (k_io == 4 * j_io + 1).astype(jnp.float32)
    maskd = jax.lax.dot_general(
        mask_full.reshape(i_q * hb_n, ws), sel,
        (((1,), (0,)), ((), ())), precision=jax.lax.Precision.DEFAULT,
        preferred_element_type=jnp.float32).reshape(i_q, hb_n * wf)

    fm = fmap_ref[0]
    partial = jax.lax.dot_general(
        maskd, fm, (((1,), (1,)), ((), ())),
        precision=jax.lax.Precision.DEFAULT,
        preferred_element_type=jnp.float32)
    cnt_blk = jnp.sum(maskd, axis=1, keepdims=True)

    @pl.when(hb == 0)
    def _init():
        out_ref[0] = partial
        cnt_ref[...] = cnt_blk

    @pl.when(hb != 0)
    def _acc():
        out_ref[0] += partial
        cnt_ref[...] += cnt_blk

    need_fb = jnp.min(cnt_ref[...]) <= 0.0

    @pl.when(need_fb)
    def _fb():
        vm = jnp.where(valid, r, -1.0)
        bmax = jnp.max(vm, axis=(1, 2))[:, None]
        h_io = jax.lax.broadcasted_iota(jnp.int32, (i_q, hb_n, ws), 1)
        gidx = ((hb * hb_n + h_io) * wf
                + jax.lax.shift_right_logical(lane - 1, 2))
        cand = jnp.where(jnp.logical_and(valid, r == bmax[:, :, None]), gidx,
                         jnp.int32(2147483647))
        fidx = jnp.min(cand, axis=(1, 2))[:, None]
        sidx = (jax.lax.broadcasted_iota(jnp.int32, (i_q, hb_n * wf), 1)
                + hb * (hb_n * wf))
        onehot = (sidx == fidx).astype(jnp.float32)
        bfeat_blk = jax.lax.dot_general(
            onehot, fm, (((1,), (1,)), ((), ())),
            precision=jax.lax.Precision.DEFAULT,
            preferred_element_type=jnp.float32)

        @pl.when(hb == 0)
        def _fb_init():
            bval_ref[...] = bmax
            bfeat_ref[...] = bfeat_blk

        @pl.when(hb != 0)
        def _fb_merge():
            upd = bmax > bval_ref[...]
            bval_ref[...] = jnp.where(upd, bmax, bval_ref[...])
            bfeat_ref[...] = jnp.where(upd, bfeat_blk, bfeat_ref[...])

    @pl.when(hb == nhb - 1)
    def _fin():
        cnt = cnt_ref[...]
        out_ref[0] = jnp.where(cnt > 0.0, out_ref[0] / cnt, bfeat_ref[...])


def kernel(features, scribbles):
    fmap = features[-1]
    b, c, hf, wf = fmap.shape
    i_q = scribbles.shape[1]
    assert scribbles.shape[2] == 4 * hf and scribbles.shape[3] == 4 * wf
    fmap_v = fmap.reshape(b, c, hf * wf)

    hb_sz = 32
    nhb = hf // hb_sz
    out = pl.pallas_call(
        functools.partial(_kern, nhb=nhb, hb_sz=hb_sz, wf=wf),
        grid=(b, nhb),
        in_specs=[
            pl.BlockSpec((1, i_q, 4 * hb_sz, 4 * wf),
                         lambda bb, hh: (bb, 0, hh, 0)),
            pl.BlockSpec((1, c, hb_sz * wf),
                         lambda bb, hh: (bb, 0, hh)),
        ],
        out_specs=pl.BlockSpec((1, i_q, c), lambda bb, hh: (bb, 0, 0)),
        out_shape=jax.ShapeDtypeStruct((b, i_q, c), jnp.float32),
        scratch_shapes=[
            pltpu.VMEM((i_q, 1), jnp.float32),
            pltpu.VMEM((i_q, 1), jnp.float32),
            pltpu.VMEM((i_q, c), jnp.float32),
        ],
    )(scribbles, fmap_v)
    return out
